# baseline (speedup 1.0000x reference)
import jax
import jax.numpy as jnp
from jax.experimental import pallas as pl

N = 10000
DIM = 128
HEADS = 2
DH = DIM // HEADS
L = 2
ALPHA = 0.2


def _elu_pallas(x):
    def body(x_ref, o_ref):
        v = x_ref[...]
        o_ref[...] = jnp.where(v > 0, v, jnp.exp(v) - 1.0)
    return pl.pallas_call(
        body,
        out_shape=jax.ShapeDtypeStruct(x.shape, x.dtype),
    )(x)


def _sp_gat(x, src, dst, W, a):
    out = x
    for l in range(L):
        heads = []
        for h in range(HEADS):
            hx = out @ W[l, h]
            e = jax.nn.leaky_relu(hx[src] @ a[l, h, :DH] + hx[dst] @ a[l, h, DH:], ALPHA)
            emax = jax.ops.segment_max(e, dst, num_segments=N)
            w = jnp.exp(e - emax[dst])
            denom = jax.ops.segment_sum(w, dst, num_segments=N)
            agg = jax.ops.segment_sum(w[:, None] * hx[src], dst, num_segments=N)
            heads.append(agg / denom[:, None])
        out = _elu_pallas(jnp.concatenate(heads, axis=1))
    return out


def kernel(ent_sr, ent_tg, rel_sr, rel_tg, W, a, edge_sr, edge_tg,
           sr_data, tg_data, h_list_sr, h_list_tg, t_list_sr, t_list_tg,
           r_list_sr, r_list_tg):
    out_sr = _sp_gat(ent_sr, edge_sr[0], edge_sr[1], W, a)
    out_tg = _sp_gat(ent_tg, edge_tg[0], edge_tg[1], W, a)
    sr_score = out_sr[h_list_sr] + rel_sr[r_list_sr] - out_sr[t_list_sr]
    tg_score = out_tg[h_list_tg] + rel_tg[r_list_tg] - out_tg[t_list_tg]
    transe_score = jnp.concatenate([sr_score, tg_score], axis=0)
    return (out_sr[sr_data], out_tg[tg_data], transe_score)


# speedup vs baseline: 13.3919x; 13.3919x over previous
import functools

import jax
import jax.numpy as jnp
from jax import lax
from jax.experimental import pallas as pl
from jax.experimental.pallas import tpu as pltpu
from jax.experimental.pallas import tpu_sc as plsc

N = 10000
NPAD = 10240
DIM = 128
DH = 64
HEADS = 2
L = 2
R = 1000
T = 100000
TPAD = 102400
B = 4096
ALPHA = 0.2

E2 = 330000
CH = 128
NCHUNK = 162
EPT = NCHUNK * CH
EP = 16 * EPT
ROWS_PT = NPAD // 16
DR = 2 * NPAD // 128
BN = 1280

_mesh = plsc.VectorSubcoreMesh(core_axis_name="c", subcore_axis_name="s",
                               num_cores=2)
_params = pltpu.CompilerParams(needs_layout_passes=False)


def _tc_project(x, w, aa):
    def body(x_ref, w_ref, a_ref, hx_ref, s_ref):
        hx = jnp.dot(x_ref[...], w_ref[...], preferred_element_type=jnp.float32)
        hx_ref[...] = hx
        s_ref[...] = jnp.dot(hx, a_ref[...], preferred_element_type=jnp.float32)
    return pl.pallas_call(
        body,
        grid=(2 * NPAD // BN,),
        in_specs=[pl.BlockSpec((BN, 128), lambda i: (i, 0)),
                  pl.BlockSpec((128, 128), lambda i: (0, 0)),
                  pl.BlockSpec((128, 8), lambda i: (0, 0))],
        out_specs=[pl.BlockSpec((BN, 128), lambda i: (i, 0)),
                   pl.BlockSpec((BN, 8), lambda i: (i, 0))],
        out_shape=[jax.ShapeDtypeStruct((2 * NPAD, 128), jnp.float32),
                   jax.ShapeDtypeStruct((2 * NPAD, 8), jnp.float32)],
    )(x, w, aa)


def _combine(ag, dn):
    x0 = ag[:, 0:64] / dn[:, 0:1]
    x1 = ag[:, 64:128] / dn[:, 1:2]
    x = jnp.concatenate([x0, x1], axis=1)
    return jnp.where(x > 0, x, jnp.exp(x) - 1.0)


def _tc_combine_project(agg, den, w, aa):
    def body(g_ref, d_ref, w_ref, a_ref, hx_ref, s_ref):
        x = _combine(g_ref[...], d_ref[...])
        hx = jnp.dot(x, w_ref[...], preferred_element_type=jnp.float32)
        hx_ref[...] = hx
        s_ref[...] = jnp.dot(hx, a_ref[...], preferred_element_type=jnp.float32)
    return pl.pallas_call(
        body,
        grid=(2 * NPAD // BN,),
        in_specs=[pl.BlockSpec((BN, 128), lambda i: (i, 0)),
                  pl.BlockSpec((BN, 2), lambda i: (i, 0)),
                  pl.BlockSpec((128, 128), lambda i: (0, 0)),
                  pl.BlockSpec((128, 8), lambda i: (0, 0))],
        out_specs=[pl.BlockSpec((BN, 128), lambda i: (i, 0)),
                   pl.BlockSpec((BN, 8), lambda i: (i, 0))],
        out_shape=[jax.ShapeDtypeStruct((2 * NPAD, 128), jnp.float32),
                   jax.ShapeDtypeStruct((2 * NPAD, 8), jnp.float32)],
    )(agg, den, w, aa)


def _tc_final(agg, den):
    def body(g_ref, d_ref, o_ref):
        o_ref[...] = _combine(g_ref[...], d_ref[...])
    return pl.pallas_call(
        body,
        grid=(2 * NPAD // BN,),
        in_specs=[pl.BlockSpec((BN, 128), lambda i: (i, 0)),
                  pl.BlockSpec((BN, 2), lambda i: (i, 0))],
        out_specs=pl.BlockSpec((BN, 128), lambda i: (i, 0)),
        out_shape=jax.ShapeDtypeStruct((2 * NPAD, 128), jnp.float32),
    )(agg, den)


@functools.partial(
    pl.kernel, mesh=_mesh, compiler_params=_params,
    out_type=[jax.ShapeDtypeStruct((4 * EP,), jnp.float32),
              jax.ShapeDtypeStruct((2 * DR, 128), jnp.float32)],
    scratch_types=[
        pltpu.VMEM((2 * NPAD,), jnp.int32),
        pltpu.VMEM((CH,), jnp.int32),
        pltpu.VMEM((CH,), jnp.int32),
        pltpu.VMEM((2 * CH,), jnp.float32),
        pltpu.VMEM((DR, 128), jnp.float32),
        pltpu.VMEM((CH,), jnp.int32),
        pltpu.VMEM((32,), jnp.int32),
        pltpu.VMEM((16,), jnp.int32),
        pltpu.VMEM((16,), jnp.float32),
        pltpu.VMEM_SHARED((DR, 128), jnp.float32),
    ],
)
def _sc_att(stp, src, dst, wout, den_out,
            stv, src_v, dst_v, wtmp, den2, iota_a, iota_b, kbuf, mbuf, denS):
    g = lax.axis_index("c")
    sid = lax.axis_index("s")
    lane = lax.iota(jnp.int32, 16)

    pltpu.sync_copy(stp.at[g], stv)

    def _zd(i, _):
        den2[i >> 3, pl.ds((i & 7) * 16, 16)] = jnp.zeros((16,), jnp.float32)
        return 0
    lax.fori_loop(0, DR * 8, _zd, 0)

    def _zi(t, _):
        iota_a[pl.ds(t * 16, 16)] = lane + t * 16
        return 0
    lax.fori_loop(0, 8, _zi, 0)
    iota_b[pl.ds(0, 16)] = lane + CH
    iota_b[pl.ds(16, 16)] = lane + CH + 16

    @pl.when(sid == 0)
    def _():
        pltpu.sync_copy(den2.at[pl.ds(0, CH)], denS.at[pl.ds(0, CH)])
        pltpu.sync_copy(den2.at[pl.ds(0, DR - CH)], denS.at[pl.ds(CH, DR - CH)])

    plsc.subcore_barrier()

    mhi = jnp.full((16,), -65536, jnp.int32)

    def _chunk(c, _):
        off = sid * EPT + c * CH
        pltpu.sync_copy(src.at[pl.ds(g * EP + off, CH)], src_v)
        pltpu.sync_copy(dst.at[pl.ds(g * EP + off, CH)], dst_v)

        for t in range(CH // 16):
            sv = src_v[pl.ds(t * 16, 16)]
            dv = dst_v[pl.ds(t * 16, 16)]
            p_s = plsc.load_gather(stv, [sv])
            p_d = plsc.load_gather(stv, [dv + NPAD])
            e0 = (plsc.bitcast(lax.shift_left(p_s, 16), jnp.float32)
                  + plsc.bitcast(lax.shift_left(p_d, 16), jnp.float32))
            e1 = (plsc.bitcast(lax.bitwise_and(p_s, mhi), jnp.float32)
                  + plsc.bitcast(lax.bitwise_and(p_d, mhi), jnp.float32))
            e0 = jnp.where(e0 >= 0, e0, ALPHA * e0)
            e1 = jnp.where(e1 >= 0, e1, ALPHA * e1)
            w0 = jnp.exp(e0)
            w1 = jnp.exp(e1)
            wtmp[pl.ds(t * 16, 16)] = w0
            wtmp[pl.ds(CH + t * 16, 16)] = w1

            k, w0s = plsc.sort_key_val(dv, w0)
            _, w1s = plsc.sort_key_val(dv, w1)
            c0 = plsc.cumsum(w0s)
            c1 = plsc.cumsum(w1s)
            kbuf[pl.ds(0, 16)] = jnp.full((16,), -1, jnp.int32)
            plsc.store_scatter(kbuf, [lane - 1], k, mask=lane >= 1)
            isend = k != kbuf[pl.ds(0, 16)]
            row = lax.shift_right_logical(k, 7)
            col = lax.bitwise_and(k, 127)
            for cs, roff in ((c0, 0), (c1, NPAD >> 7)):
                m = jnp.where(isend, cs, 0.0)
                mbuf[pl.ds(0, 16)] = jnp.zeros((16,), jnp.float32)
                plsc.store_scatter(mbuf, [lane + 1], m, mask=lane <= 14)
                pe = plsc.cummax(mbuf[pl.ds(0, 16)])
                plsc.addupdate_scatter(den2, [row + roff, col], cs - pe,
                                       mask=isend)

        pltpu.sync_copy(wtmp.at[pl.ds(0, CH)],
                        wout.at[pl.ds(2 * g * EP + off, CH)])
        pltpu.sync_copy(wtmp.at[pl.ds(CH, CH)],
                        wout.at[pl.ds((2 * g + 1) * EP + off, CH)])
        return 0

    lax.fori_loop(0, NCHUNK, _chunk, 0)

    pltpu.sync_copy(den2.at[pl.ds(0, CH)], denS.at[iota_a], add=True)
    pltpu.sync_copy(den2.at[pl.ds(CH, DR - CH)], denS.at[iota_b], add=True)
    plsc.subcore_barrier()

    @pl.when(sid < 10)
    def _():
        pltpu.sync_copy(denS.at[pl.ds(sid * 16, 16)],
                        den_out.at[pl.ds(g * DR + sid * 16, 16)])


@functools.partial(
    pl.kernel, mesh=_mesh, compiler_params=_params,
    out_type=jax.ShapeDtypeStruct((2 * NPAD, 128), jnp.float32),
    scratch_types=[
        pltpu.VMEM((CH,), jnp.int32),
        pltpu.VMEM((CH,), jnp.int32),
        pltpu.VMEM((CH,), jnp.float32),
        pltpu.VMEM((CH,), jnp.float32),
        pltpu.VMEM((CH, 128), jnp.float32),
        pltpu.VMEM((CH, 128), jnp.float32),
        pltpu.VMEM_SHARED((NPAD, 128), jnp.float32),
        pltpu.SemaphoreType.DMA,
    ],
)
def _sc_agg(hx2, srcoff, dst, wall, agg_out,
            soff_v, dst_v, wv0, wv1, gbuf, zbuf, acc, gsem):
    g = lax.axis_index("c")
    sid = lax.axis_index("s")

    def _z(i, _):
        zbuf[i >> 3, pl.ds((i & 7) * 16, 16)] = jnp.zeros((16,), jnp.float32)
        return 0
    lax.fori_loop(0, CH * 8, _z, 0)

    def _zcopy(j, _):
        pltpu.sync_copy(zbuf, acc.at[pl.ds(sid * ROWS_PT + j * CH, CH)])
        return 0
    lax.fori_loop(0, ROWS_PT // CH, _zcopy, 0)
    plsc.subcore_barrier()

    def _chunk(c, _):
        off = sid * EPT + c * CH
        pltpu.sync_copy(srcoff.at[pl.ds(g * EP + off, CH)], soff_v)
        pltpu.sync_copy(dst.at[pl.ds(g * EP + off, CH)], dst_v)
        cp = pltpu.async_copy(hx2.at[soff_v], gbuf, gsem)
        pltpu.sync_copy(wall.at[pl.ds(2 * g * EP + off, CH)], wv0)
        pltpu.sync_copy(wall.at[pl.ds((2 * g + 1) * EP + off, CH)], wv1)
        cp.wait()

        def _mul(e, _):
            w0 = plsc.load_gather(wv0, [jnp.full((16,), e, jnp.int32)])
            w1 = plsc.load_gather(wv1, [jnp.full((16,), e, jnp.int32)])
            for d in range(4):
                sl = pl.ds(d * 16, 16)
                gbuf[e, sl] = gbuf[e, sl] * w0
            for d in range(4, 8):
                sl = pl.ds(d * 16, 16)
                gbuf[e, sl] = gbuf[e, sl] * w1
            return 0
        lax.fori_loop(0, CH, _mul, 0)

        pltpu.sync_copy(gbuf, acc.at[dst_v], add=True)
        return 0

    lax.fori_loop(0, NCHUNK, _chunk, 0)
    plsc.subcore_barrier()

    pltpu.sync_copy(acc.at[pl.ds(sid * ROWS_PT, ROWS_PT)],
                    agg_out.at[pl.ds(g * NPAD + sid * ROWS_PT, ROWS_PT)])


_TPT = TPAD // 16
_BPT = B // 16


@functools.partial(
    pl.kernel, mesh=_mesh, compiler_params=_params,
    out_type=[jax.ShapeDtypeStruct((2 * TPAD, 128), jnp.float32),
              jax.ShapeDtypeStruct((2 * B, 128), jnp.float32)],
    scratch_types=[
        pltpu.VMEM((CH,), jnp.int32),
        pltpu.VMEM((CH,), jnp.int32),
        pltpu.VMEM((CH,), jnp.int32),
        pltpu.VMEM((CH, 128), jnp.float32),
        pltpu.VMEM((CH, 128), jnp.float32),
        pltpu.VMEM((CH, 128), jnp.float32),
        pltpu.SemaphoreType.DMA,
        pltpu.SemaphoreType.DMA,
        pltpu.SemaphoreType.DMA,
    ],
)
def _sc_score(out2, rel2, ho, ro, to, bo, tr, bout,
              ho_v, ro_v, to_v, hb, rb, tb, s1, s2, s3):
    g = lax.axis_index("c")
    sid = lax.axis_index("s")
    tbase = g * TPAD + sid * _TPT

    def _fuse(e, _):
        for d in range(8):
            sl = pl.ds(d * 16, 16)
            hb[e, sl] = hb[e, sl] + rb[e, sl] - tb[e, sl]
        return 0

    def _chunk(c, _):
        off = tbase + c * CH
        pltpu.sync_copy(ho.at[pl.ds(off, CH)], ho_v)
        pltpu.sync_copy(ro.at[pl.ds(off, CH)], ro_v)
        pltpu.sync_copy(to.at[pl.ds(off, CH)], to_v)
        c1 = pltpu.async_copy(out2.at[ho_v], hb, s1)
        c2 = pltpu.async_copy(rel2.at[ro_v], rb, s2)
        c3 = pltpu.async_copy(out2.at[to_v], tb, s3)
        c1.wait(); c2.wait(); c3.wait()
        lax.fori_loop(0, CH, _fuse, 0)
        pltpu.sync_copy(hb, tr.at[pl.ds(off, CH)])
        return 0
    lax.fori_loop(0, _TPT // CH, _chunk, 0)

    bbase = g * B + sid * _BPT

    def _bchunk(c, _):
        off = bbase + c * CH
        pltpu.sync_copy(bo.at[pl.ds(off, CH)], ho_v)
        pltpu.async_copy(out2.at[ho_v], hb, s1).wait()
        pltpu.sync_copy(hb, bout.at[pl.ds(off, CH)])
        return 0
    lax.fori_loop(0, _BPT // CH, _bchunk, 0)


def _pad_idx(x, total, spread):
    x = x.astype(jnp.int32)
    npad = total - x.shape[0]
    if spread:
        fill = N + (jnp.arange(npad, dtype=jnp.int32) % (NPAD - N))
    else:
        fill = jnp.zeros((npad,), jnp.int32)
    return jnp.concatenate([x, fill])


def _pack_logits(s):
    u = lax.bitcast_convert_type(s.astype(jnp.bfloat16), jnp.uint16)
    u = u.astype(jnp.uint32)
    psrc = (u[:, 0] | (u[:, 1] << 16)).astype(jnp.int32)
    pdst = (u[:, 2] | (u[:, 3] << 16)).astype(jnp.int32)
    psrc = psrc.reshape(2, NPAD)
    pdst = pdst.reshape(2, NPAD)
    return jnp.concatenate([psrc, pdst], axis=1)


def kernel(ent_sr, ent_tg, rel_sr, rel_tg, W, a, edge_sr, edge_tg,
           sr_data, tg_data, h_list_sr, h_list_tg, t_list_sr, t_list_tg,
           r_list_sr, r_list_tg):
    f32 = jnp.float32
    x0 = jnp.stack([jnp.pad(ent_sr.astype(f32), ((0, NPAD - N), (0, 0))),
                    jnp.pad(ent_tg.astype(f32), ((0, NPAD - N), (0, 0)))])
    x0 = x0.reshape(2 * NPAD, 128)

    Wcat = jnp.concatenate([W[:, 0], W[:, 1]], axis=-1).astype(f32)
    A = jnp.zeros((L, 128, 8), f32)
    A = A.at[:, :64, 0].set(a[:, 0, :64]).at[:, 64:, 1].set(a[:, 1, :64])
    A = A.at[:, :64, 2].set(a[:, 0, 64:]).at[:, 64:, 3].set(a[:, 1, 64:])

    goff = (jnp.arange(2, dtype=jnp.int32) * NPAD)[:, None]
    src2 = jnp.stack([_pad_idx(edge_sr[0], EP, True),
                      _pad_idx(edge_tg[0], EP, True)])
    dst2 = jnp.stack([_pad_idx(edge_sr[1], EP, True),
                      _pad_idx(edge_tg[1], EP, True)])
    srcoff = (src2 + goff).reshape(-1)
    src_f = src2.reshape(-1)
    dst_f = dst2.reshape(-1)

    def _den_t(d):
        return jnp.transpose(d.reshape(2, 2, NPAD), (0, 2, 1)).reshape(2 * NPAD, 2)

    hx, s = _tc_project(x0, Wcat[0], A[0])
    w01, den = _sc_att(_pack_logits(s), src_f, dst_f)
    agg = _sc_agg(hx, srcoff, dst_f, w01)

    hx, s = _tc_combine_project(agg, _den_t(den), Wcat[1], A[1])
    w01, den = _sc_att(_pack_logits(s), src_f, dst_f)
    agg = _sc_agg(hx, srcoff, dst_f, w01)

    out2 = _tc_final(agg, _den_t(den))

    rel2 = jnp.concatenate([rel_sr, rel_tg], axis=0).astype(f32)
    ho = jnp.concatenate([_pad_idx(h_list_sr, TPAD, False),
                          _pad_idx(h_list_tg, TPAD, False) + NPAD])
    to = jnp.concatenate([_pad_idx(t_list_sr, TPAD, False),
                          _pad_idx(t_list_tg, TPAD, False) + NPAD])
    ro = jnp.concatenate([_pad_idx(r_list_sr, TPAD, False),
                          _pad_idx(r_list_tg, TPAD, False) + R])
    bo = jnp.concatenate([sr_data.astype(jnp.int32),
                          tg_data.astype(jnp.int32) + NPAD])

    tr, bout = _sc_score(out2, rel2, ho, ro, to, bo)
    transe = jnp.concatenate([tr[:T], tr[TPAD:TPAD + T]], axis=0)
    return (bout[:B], bout[B:], transe)


# speedup vs baseline: 13.9647x; 1.0428x over previous
import functools

import jax
import jax.numpy as jnp
from jax import lax
from jax.experimental import pallas as pl
from jax.experimental.pallas import tpu as pltpu
from jax.experimental.pallas import tpu_sc as plsc

N = 10000
NPAD = 10240
DIM = 128
DH = 64
HEADS = 2
L = 2
R = 1000
T = 100000
TPAD = 102400
B = 4096
ALPHA = 0.2

E2 = 330000
CH = 128
NCHUNK = 162
EPT = NCHUNK * CH
EP = 16 * EPT
ROWS_PT = NPAD // 16
DR = 2 * NPAD // 128
BN = 1280

_mesh = plsc.VectorSubcoreMesh(core_axis_name="c", subcore_axis_name="s",
                               num_cores=2)
_params = pltpu.CompilerParams(needs_layout_passes=False)


def _tc_project(x, w, aa):
    def body(x_ref, w_ref, a_ref, hx_ref, s_ref):
        hx = jnp.dot(x_ref[...], w_ref[...], preferred_element_type=jnp.float32)
        hx_ref[...] = hx
        s_ref[...] = jnp.dot(hx, a_ref[...], preferred_element_type=jnp.float32)
    return pl.pallas_call(
        body,
        grid=(2 * NPAD // BN,),
        in_specs=[pl.BlockSpec((BN, 128), lambda i: (i, 0)),
                  pl.BlockSpec((128, 128), lambda i: (0, 0)),
                  pl.BlockSpec((128, 8), lambda i: (0, 0))],
        out_specs=[pl.BlockSpec((BN, 128), lambda i: (i, 0)),
                   pl.BlockSpec((BN, 8), lambda i: (i, 0))],
        out_shape=[jax.ShapeDtypeStruct((2 * NPAD, 128), jnp.float32),
                   jax.ShapeDtypeStruct((2 * NPAD, 8), jnp.float32)],
    )(x, w, aa)


def _combine(ag, dn):
    x0 = ag[:, 0:64] / dn[:, 0:1]
    x1 = ag[:, 64:128] / dn[:, 1:2]
    x = jnp.concatenate([x0, x1], axis=1)
    return jnp.where(x > 0, x, jnp.exp(x) - 1.0)


def _tc_combine_project(agg, den, w, aa):
    def body(g_ref, d_ref, w_ref, a_ref, hx_ref, s_ref):
        x = _combine(g_ref[...], d_ref[...])
        hx = jnp.dot(x, w_ref[...], preferred_element_type=jnp.float32)
        hx_ref[...] = hx
        s_ref[...] = jnp.dot(hx, a_ref[...], preferred_element_type=jnp.float32)
    return pl.pallas_call(
        body,
        grid=(2 * NPAD // BN,),
        in_specs=[pl.BlockSpec((BN, 128), lambda i: (i, 0)),
                  pl.BlockSpec((BN, 2), lambda i: (i, 0)),
                  pl.BlockSpec((128, 128), lambda i: (0, 0)),
                  pl.BlockSpec((128, 8), lambda i: (0, 0))],
        out_specs=[pl.BlockSpec((BN, 128), lambda i: (i, 0)),
                   pl.BlockSpec((BN, 8), lambda i: (i, 0))],
        out_shape=[jax.ShapeDtypeStruct((2 * NPAD, 128), jnp.float32),
                   jax.ShapeDtypeStruct((2 * NPAD, 8), jnp.float32)],
    )(agg, den, w, aa)


def _tc_final(agg, den):
    def body(g_ref, d_ref, o_ref):
        o_ref[...] = _combine(g_ref[...], d_ref[...])
    return pl.pallas_call(
        body,
        grid=(2 * NPAD // BN,),
        in_specs=[pl.BlockSpec((BN, 128), lambda i: (i, 0)),
                  pl.BlockSpec((BN, 2), lambda i: (i, 0))],
        out_specs=pl.BlockSpec((BN, 128), lambda i: (i, 0)),
        out_shape=jax.ShapeDtypeStruct((2 * NPAD, 128), jnp.float32),
    )(agg, den)


@functools.partial(
    pl.kernel, mesh=_mesh, compiler_params=_params,
    out_type=[jax.ShapeDtypeStruct((4 * EP,), jnp.float32),
              jax.ShapeDtypeStruct((2 * DR, 128), jnp.float32)],
    scratch_types=[
        pltpu.VMEM((2 * NPAD,), jnp.int32),
        pltpu.VMEM((CH,), jnp.int32),
        pltpu.VMEM((CH,), jnp.int32),
        pltpu.VMEM((2 * CH,), jnp.float32),
        pltpu.VMEM((DR, 128), jnp.float32),
        pltpu.VMEM((CH,), jnp.int32),
        pltpu.VMEM((32,), jnp.int32),
        pltpu.VMEM((16,), jnp.int32),
        pltpu.VMEM((16,), jnp.float32),
        pltpu.VMEM_SHARED((DR, 128), jnp.float32),
    ],
)
def _sc_att(stp, src, dst, wout, den_out,
            stv, src_v, dst_v, wtmp, den2, iota_a, iota_b, kbuf, mbuf, denS):
    g = lax.axis_index("c")
    sid = lax.axis_index("s")
    lane = lax.iota(jnp.int32, 16)

    pltpu.sync_copy(stp.at[g], stv)

    def _zd(i, _):
        den2[i >> 3, pl.ds((i & 7) * 16, 16)] = jnp.zeros((16,), jnp.float32)
        return 0
    lax.fori_loop(0, DR * 8, _zd, 0)

    def _zi(t, _):
        iota_a[pl.ds(t * 16, 16)] = lane + t * 16
        return 0
    lax.fori_loop(0, 8, _zi, 0)
    iota_b[pl.ds(0, 16)] = lane + CH
    iota_b[pl.ds(16, 16)] = lane + CH + 16

    @pl.when(sid == 0)
    def _():
        pltpu.sync_copy(den2.at[pl.ds(0, CH)], denS.at[pl.ds(0, CH)])
        pltpu.sync_copy(den2.at[pl.ds(0, DR - CH)], denS.at[pl.ds(CH, DR - CH)])

    plsc.subcore_barrier()

    mhi = jnp.full((16,), -65536, jnp.int32)

    def _chunk(c, _):
        off = sid * EPT + c * CH
        pltpu.sync_copy(src.at[pl.ds(g * EP + off, CH)], src_v)
        pltpu.sync_copy(dst.at[pl.ds(g * EP + off, CH)], dst_v)

        for t in range(CH // 16):
            sv = src_v[pl.ds(t * 16, 16)]
            dv = dst_v[pl.ds(t * 16, 16)]
            p_s = plsc.load_gather(stv, [sv])
            p_d = plsc.load_gather(stv, [dv + NPAD])
            e0 = (plsc.bitcast(lax.shift_left(p_s, 16), jnp.float32)
                  + plsc.bitcast(lax.shift_left(p_d, 16), jnp.float32))
            e1 = (plsc.bitcast(lax.bitwise_and(p_s, mhi), jnp.float32)
                  + plsc.bitcast(lax.bitwise_and(p_d, mhi), jnp.float32))
            e0 = jnp.where(e0 >= 0, e0, ALPHA * e0)
            e1 = jnp.where(e1 >= 0, e1, ALPHA * e1)
            w0 = jnp.exp(e0)
            w1 = jnp.exp(e1)
            wtmp[pl.ds(t * 16, 16)] = w0
            wtmp[pl.ds(CH + t * 16, 16)] = w1

            k, w0s = plsc.sort_key_val(dv, w0)
            _, w1s = plsc.sort_key_val(dv, w1)
            c0 = plsc.cumsum(w0s)
            c1 = plsc.cumsum(w1s)
            kbuf[pl.ds(0, 16)] = jnp.full((16,), -1, jnp.int32)
            plsc.store_scatter(kbuf, [lane - 1], k, mask=lane >= 1)
            isend = k != kbuf[pl.ds(0, 16)]
            row = lax.shift_right_logical(k, 7)
            col = lax.bitwise_and(k, 127)
            for cs, roff in ((c0, 0), (c1, NPAD >> 7)):
                m = jnp.where(isend, cs, 0.0)
                mbuf[pl.ds(0, 16)] = jnp.zeros((16,), jnp.float32)
                plsc.store_scatter(mbuf, [lane + 1], m, mask=lane <= 14)
                pe = plsc.cummax(mbuf[pl.ds(0, 16)])
                plsc.addupdate_scatter(den2, [row + roff, col], cs - pe,
                                       mask=isend)

        pltpu.sync_copy(wtmp.at[pl.ds(0, CH)],
                        wout.at[pl.ds(2 * g * EP + off, CH)])
        pltpu.sync_copy(wtmp.at[pl.ds(CH, CH)],
                        wout.at[pl.ds((2 * g + 1) * EP + off, CH)])
        return 0

    lax.fori_loop(0, NCHUNK, _chunk, 0)

    pltpu.sync_copy(den2.at[pl.ds(0, CH)], denS.at[iota_a], add=True)
    pltpu.sync_copy(den2.at[pl.ds(CH, DR - CH)], denS.at[iota_b], add=True)
    plsc.subcore_barrier()

    @pl.when(sid < 10)
    def _():
        pltpu.sync_copy(denS.at[pl.ds(sid * 16, 16)],
                        den_out.at[pl.ds(g * DR + sid * 16, 16)])


@functools.partial(
    pl.kernel, mesh=_mesh, compiler_params=_params,
    out_type=jax.ShapeDtypeStruct((2 * NPAD, 128), jnp.float32),
    scratch_types=[
        pltpu.VMEM((CH,), jnp.int32),
        pltpu.VMEM((CH,), jnp.int32),
        pltpu.VMEM((CH,), jnp.float32),
        pltpu.VMEM((CH,), jnp.float32),
        pltpu.VMEM((CH, 128), jnp.float32),
        pltpu.VMEM((CH, 128), jnp.float32),
        pltpu.VMEM_SHARED((NPAD, 128), jnp.float32),
        pltpu.SemaphoreType.DMA,
    ],
)
def _sc_agg(hx2, srcoff, dst, wall, agg_out,
            soff_v, dst_v, wv0, wv1, gbuf, zbuf, acc, gsem):
    g = lax.axis_index("c")
    sid = lax.axis_index("s")

    def _z(i, _):
        zbuf[i >> 3, pl.ds((i & 7) * 16, 16)] = jnp.zeros((16,), jnp.float32)
        return 0
    lax.fori_loop(0, CH * 8, _z, 0)

    def _zcopy(j, _):
        pltpu.sync_copy(zbuf, acc.at[pl.ds(sid * ROWS_PT + j * CH, CH)])
        return 0
    lax.fori_loop(0, ROWS_PT // CH, _zcopy, 0)
    plsc.subcore_barrier()

    def _chunk(c, _):
        off = sid * EPT + c * CH
        pltpu.sync_copy(srcoff.at[pl.ds(g * EP + off, CH)], soff_v)
        pltpu.sync_copy(dst.at[pl.ds(g * EP + off, CH)], dst_v)
        cp = pltpu.async_copy(hx2.at[soff_v], gbuf, gsem)
        pltpu.sync_copy(wall.at[pl.ds(2 * g * EP + off, CH)], wv0)
        pltpu.sync_copy(wall.at[pl.ds((2 * g + 1) * EP + off, CH)], wv1)
        cp.wait()

        @plsc.parallel_loop(0, CH, 1, unroll=8)
        def _mul(e):
            w0 = plsc.load_gather(wv0, [jnp.full((16,), e, jnp.int32)])
            w1 = plsc.load_gather(wv1, [jnp.full((16,), e, jnp.int32)])
            for d in range(4):
                sl = pl.ds(d * 16, 16)
                gbuf[e, sl] = gbuf[e, sl] * w0
            for d in range(4, 8):
                sl = pl.ds(d * 16, 16)
                gbuf[e, sl] = gbuf[e, sl] * w1

        pltpu.sync_copy(gbuf, acc.at[dst_v], add=True)
        return 0

    lax.fori_loop(0, NCHUNK, _chunk, 0)
    plsc.subcore_barrier()

    pltpu.sync_copy(acc.at[pl.ds(sid * ROWS_PT, ROWS_PT)],
                    agg_out.at[pl.ds(g * NPAD + sid * ROWS_PT, ROWS_PT)])


_TPT = TPAD // 16
_BPT = B // 16


@functools.partial(
    pl.kernel, mesh=_mesh, compiler_params=_params,
    out_type=[jax.ShapeDtypeStruct((2 * TPAD, 128), jnp.float32),
              jax.ShapeDtypeStruct((2 * B, 128), jnp.float32)],
    scratch_types=[
        pltpu.VMEM((CH,), jnp.int32),
        pltpu.VMEM((CH,), jnp.int32),
        pltpu.VMEM((CH,), jnp.int32),
        pltpu.VMEM((CH, 128), jnp.float32),
        pltpu.VMEM((CH, 128), jnp.float32),
        pltpu.VMEM((CH, 128), jnp.float32),
        pltpu.SemaphoreType.DMA,
        pltpu.SemaphoreType.DMA,
        pltpu.SemaphoreType.DMA,
    ],
)
def _sc_score(out2, rel2, ho, ro, to, bo, tr, bout,
              ho_v, ro_v, to_v, hb, rb, tb, s1, s2, s3):
    g = lax.axis_index("c")
    sid = lax.axis_index("s")
    tbase = g * TPAD + sid * _TPT

    def _chunk(c, _):
        off = tbase + c * CH
        pltpu.sync_copy(ho.at[pl.ds(off, CH)], ho_v)
        pltpu.sync_copy(ro.at[pl.ds(off, CH)], ro_v)
        pltpu.sync_copy(to.at[pl.ds(off, CH)], to_v)
        c1 = pltpu.async_copy(out2.at[ho_v], hb, s1)
        c2 = pltpu.async_copy(rel2.at[ro_v], rb, s2)
        c3 = pltpu.async_copy(out2.at[to_v], tb, s3)
        c1.wait(); c2.wait(); c3.wait()

        @plsc.parallel_loop(0, CH, 1, unroll=8)
        def _fuse(e):
            for d in range(8):
                sl = pl.ds(d * 16, 16)
                hb[e, sl] = hb[e, sl] + rb[e, sl] - tb[e, sl]
        pltpu.sync_copy(hb, tr.at[pl.ds(off, CH)])
        return 0
    lax.fori_loop(0, _TPT // CH, _chunk, 0)

    bbase = g * B + sid * _BPT

    def _bchunk(c, _):
        off = bbase + c * CH
        pltpu.sync_copy(bo.at[pl.ds(off, CH)], ho_v)
        pltpu.async_copy(out2.at[ho_v], hb, s1).wait()
        pltpu.sync_copy(hb, bout.at[pl.ds(off, CH)])
        return 0
    lax.fori_loop(0, _BPT // CH, _bchunk, 0)


def _pad_idx(x, total, spread):
    x = x.astype(jnp.int32)
    npad = total - x.shape[0]
    if spread:
        fill = N + (jnp.arange(npad, dtype=jnp.int32) % (NPAD - N))
    else:
        fill = jnp.zeros((npad,), jnp.int32)
    return jnp.concatenate([x, fill])


def _pack_logits(s):
    u = lax.bitcast_convert_type(s.astype(jnp.bfloat16), jnp.uint16)
    u = u.astype(jnp.uint32)
    psrc = (u[:, 0] | (u[:, 1] << 16)).astype(jnp.int32)
    pdst = (u[:, 2] | (u[:, 3] << 16)).astype(jnp.int32)
    psrc = psrc.reshape(2, NPAD)
    pdst = pdst.reshape(2, NPAD)
    return jnp.concatenate([psrc, pdst], axis=1)


def kernel(ent_sr, ent_tg, rel_sr, rel_tg, W, a, edge_sr, edge_tg,
           sr_data, tg_data, h_list_sr, h_list_tg, t_list_sr, t_list_tg,
           r_list_sr, r_list_tg):
    f32 = jnp.float32
    x0 = jnp.stack([jnp.pad(ent_sr.astype(f32), ((0, NPAD - N), (0, 0))),
                    jnp.pad(ent_tg.astype(f32), ((0, NPAD - N), (0, 0)))])
    x0 = x0.reshape(2 * NPAD, 128)

    Wcat = jnp.concatenate([W[:, 0], W[:, 1]], axis=-1).astype(f32)
    A = jnp.zeros((L, 128, 8), f32)
    A = A.at[:, :64, 0].set(a[:, 0, :64]).at[:, 64:, 1].set(a[:, 1, :64])
    A = A.at[:, :64, 2].set(a[:, 0, 64:]).at[:, 64:, 3].set(a[:, 1, 64:])

    goff = (jnp.arange(2, dtype=jnp.int32) * NPAD)[:, None]
    src2 = jnp.stack([_pad_idx(edge_sr[0], EP, True),
                      _pad_idx(edge_tg[0], EP, True)])
    dst2 = jnp.stack([_pad_idx(edge_sr[1], EP, True),
                      _pad_idx(edge_tg[1], EP, True)])
    srcoff = (src2 + goff).reshape(-1)
    src_f = src2.reshape(-1)
    dst_f = dst2.reshape(-1)

    def _den_t(d):
        return jnp.transpose(d.reshape(2, 2, NPAD), (0, 2, 1)).reshape(2 * NPAD, 2)

    hx, s = _tc_project(x0, Wcat[0], A[0])
    w01, den = _sc_att(_pack_logits(s), src_f, dst_f)
    agg = _sc_agg(hx, srcoff, dst_f, w01)

    hx, s = _tc_combine_project(agg, _den_t(den), Wcat[1], A[1])
    w01, den = _sc_att(_pack_logits(s), src_f, dst_f)
    agg = _sc_agg(hx, srcoff, dst_f, w01)

    out2 = _tc_final(agg, _den_t(den))

    rel2 = jnp.concatenate([rel_sr, rel_tg], axis=0).astype(f32)
    ho = jnp.concatenate([_pad_idx(h_list_sr, TPAD, False),
                          _pad_idx(h_list_tg, TPAD, False) + NPAD])
    to = jnp.concatenate([_pad_idx(t_list_sr, TPAD, False),
                          _pad_idx(t_list_tg, TPAD, False) + NPAD])
    ro = jnp.concatenate([_pad_idx(r_list_sr, TPAD, False),
                          _pad_idx(r_list_tg, TPAD, False) + R])
    bo = jnp.concatenate([sr_data.astype(jnp.int32),
                          tg_data.astype(jnp.int32) + NPAD])

    tr, bout = _sc_score(out2, rel2, ho, ro, to, bo)
    transe = jnp.concatenate([tr[:T], tr[TPAD:TPAD + T]], axis=0)
    return (bout[:B], bout[B:], transe)


# speedup vs baseline: 14.8760x; 1.0653x over previous
import functools

import jax
import jax.numpy as jnp
from jax import lax
from jax.experimental import pallas as pl
from jax.experimental.pallas import tpu as pltpu
from jax.experimental.pallas import tpu_sc as plsc

N = 10000
NPAD = 10240
DIM = 128
DH = 64
HEADS = 2
L = 2
R = 1000
T = 100000
TPAD = 102400
B = 4096
ALPHA = 0.2

E2 = 330000
CH = 128
NCHUNK = 162
EPT = NCHUNK * CH
EP = 16 * EPT
ROWS_PT = NPAD // 16
DR = 2 * NPAD // 128
BN = 1280

_mesh = plsc.VectorSubcoreMesh(core_axis_name="c", subcore_axis_name="s",
                               num_cores=2)
_params = pltpu.CompilerParams(needs_layout_passes=False)


def _tc_project(x, w, aa):
    def body(x_ref, w_ref, a_ref, hx_ref, s_ref):
        hx = jnp.dot(x_ref[...], w_ref[...], preferred_element_type=jnp.float32)
        hx_ref[...] = hx
        s_ref[...] = jnp.dot(hx, a_ref[...], preferred_element_type=jnp.float32)
    return pl.pallas_call(
        body,
        grid=(2 * NPAD // BN,),
        in_specs=[pl.BlockSpec((BN, 128), lambda i: (i, 0)),
                  pl.BlockSpec((128, 128), lambda i: (0, 0)),
                  pl.BlockSpec((128, 8), lambda i: (0, 0))],
        out_specs=[pl.BlockSpec((BN, 128), lambda i: (i, 0)),
                   pl.BlockSpec((BN, 8), lambda i: (i, 0))],
        out_shape=[jax.ShapeDtypeStruct((2 * NPAD, 128), jnp.float32),
                   jax.ShapeDtypeStruct((2 * NPAD, 8), jnp.float32)],
    )(x, w, aa)


def _combine(ag, dn):
    x0 = ag[:, 0:64] / dn[:, 0:1]
    x1 = ag[:, 64:128] / dn[:, 1:2]
    x = jnp.concatenate([x0, x1], axis=1)
    return jnp.where(x > 0, x, jnp.exp(x) - 1.0)


def _tc_combine_project(agg, den, w, aa):
    def body(g_ref, d_ref, w_ref, a_ref, hx_ref, s_ref):
        x = _combine(g_ref[...], d_ref[...])
        hx = jnp.dot(x, w_ref[...], preferred_element_type=jnp.float32)
        hx_ref[...] = hx
        s_ref[...] = jnp.dot(hx, a_ref[...], preferred_element_type=jnp.float32)
    return pl.pallas_call(
        body,
        grid=(2 * NPAD // BN,),
        in_specs=[pl.BlockSpec((BN, 128), lambda i: (i, 0)),
                  pl.BlockSpec((BN, 2), lambda i: (i, 0)),
                  pl.BlockSpec((128, 128), lambda i: (0, 0)),
                  pl.BlockSpec((128, 8), lambda i: (0, 0))],
        out_specs=[pl.BlockSpec((BN, 128), lambda i: (i, 0)),
                   pl.BlockSpec((BN, 8), lambda i: (i, 0))],
        out_shape=[jax.ShapeDtypeStruct((2 * NPAD, 128), jnp.float32),
                   jax.ShapeDtypeStruct((2 * NPAD, 8), jnp.float32)],
    )(agg, den, w, aa)


def _tc_final(agg, den):
    def body(g_ref, d_ref, o_ref):
        o_ref[...] = _combine(g_ref[...], d_ref[...])
    return pl.pallas_call(
        body,
        grid=(2 * NPAD // BN,),
        in_specs=[pl.BlockSpec((BN, 128), lambda i: (i, 0)),
                  pl.BlockSpec((BN, 2), lambda i: (i, 0))],
        out_specs=pl.BlockSpec((BN, 128), lambda i: (i, 0)),
        out_shape=jax.ShapeDtypeStruct((2 * NPAD, 128), jnp.float32),
    )(agg, den)


@functools.partial(
    pl.kernel, mesh=_mesh, compiler_params=_params,
    out_type=[jax.ShapeDtypeStruct((4 * EP,), jnp.float32),
              jax.ShapeDtypeStruct((2 * DR, 128), jnp.float32)],
    scratch_types=[
        pltpu.VMEM((2 * NPAD,), jnp.int32),
        pltpu.VMEM((CH,), jnp.int32),
        pltpu.VMEM((CH,), jnp.int32),
        pltpu.VMEM((2 * CH,), jnp.float32),
        pltpu.VMEM((DR, 128), jnp.float32),
        pltpu.VMEM((CH,), jnp.int32),
        pltpu.VMEM((32,), jnp.int32),
        pltpu.VMEM((16,), jnp.int32),
        pltpu.VMEM((16,), jnp.float32),
        pltpu.VMEM_SHARED((DR, 128), jnp.float32),
    ],
)
def _sc_att(stp, src, dst, wout, den_out,
            stv, src_v, dst_v, wtmp, den2, iota_a, iota_b, kbuf, mbuf, denS):
    g = lax.axis_index("c")
    sid = lax.axis_index("s")
    lane = lax.iota(jnp.int32, 16)

    pltpu.sync_copy(stp.at[g], stv)

    def _zd(i, _):
        den2[i >> 3, pl.ds((i & 7) * 16, 16)] = jnp.zeros((16,), jnp.float32)
        return 0
    lax.fori_loop(0, DR * 8, _zd, 0)

    def _zi(t, _):
        iota_a[pl.ds(t * 16, 16)] = lane + t * 16
        return 0
    lax.fori_loop(0, 8, _zi, 0)
    iota_b[pl.ds(0, 16)] = lane + CH
    iota_b[pl.ds(16, 16)] = lane + CH + 16

    @pl.when(sid == 0)
    def _():
        pltpu.sync_copy(den2.at[pl.ds(0, CH)], denS.at[pl.ds(0, CH)])
        pltpu.sync_copy(den2.at[pl.ds(0, DR - CH)], denS.at[pl.ds(CH, DR - CH)])

    plsc.subcore_barrier()

    mhi = jnp.full((16,), -65536, jnp.int32)

    def _chunk(c, _):
        off = sid * EPT + c * CH
        pltpu.sync_copy(src.at[pl.ds(g * EP + off, CH)], src_v)
        pltpu.sync_copy(dst.at[pl.ds(g * EP + off, CH)], dst_v)

        for t in range(CH // 16):
            sv = src_v[pl.ds(t * 16, 16)]
            dv = dst_v[pl.ds(t * 16, 16)]
            p_s = plsc.load_gather(stv, [sv])
            p_d = plsc.load_gather(stv, [dv + NPAD])
            e0 = (plsc.bitcast(lax.shift_left(p_s, 16), jnp.float32)
                  + plsc.bitcast(lax.shift_left(p_d, 16), jnp.float32))
            e1 = (plsc.bitcast(lax.bitwise_and(p_s, mhi), jnp.float32)
                  + plsc.bitcast(lax.bitwise_and(p_d, mhi), jnp.float32))
            e0 = jnp.where(e0 >= 0, e0, ALPHA * e0)
            e1 = jnp.where(e1 >= 0, e1, ALPHA * e1)
            w0 = jnp.exp(e0)
            w1 = jnp.exp(e1)
            wtmp[pl.ds(t * 16, 16)] = w0
            wtmp[pl.ds(CH + t * 16, 16)] = w1

            k, w0s = plsc.sort_key_val(dv, w0)
            _, w1s = plsc.sort_key_val(dv, w1)
            c0 = plsc.cumsum(w0s)
            c1 = plsc.cumsum(w1s)
            kbuf[pl.ds(0, 16)] = jnp.full((16,), -1, jnp.int32)
            plsc.store_scatter(kbuf, [lane - 1], k, mask=lane >= 1)
            isend = k != kbuf[pl.ds(0, 16)]
            row = lax.shift_right_logical(k, 7)
            col = lax.bitwise_and(k, 127)
            for cs, roff in ((c0, 0), (c1, NPAD >> 7)):
                m = jnp.where(isend, cs, 0.0)
                mbuf[pl.ds(0, 16)] = jnp.zeros((16,), jnp.float32)
                plsc.store_scatter(mbuf, [lane + 1], m, mask=lane <= 14)
                pe = plsc.cummax(mbuf[pl.ds(0, 16)])
                plsc.addupdate_scatter(den2, [row + roff, col], cs - pe,
                                       mask=isend)

        pltpu.sync_copy(wtmp.at[pl.ds(0, CH)],
                        wout.at[pl.ds(2 * g * EP + off, CH)])
        pltpu.sync_copy(wtmp.at[pl.ds(CH, CH)],
                        wout.at[pl.ds((2 * g + 1) * EP + off, CH)])
        return 0

    lax.fori_loop(0, NCHUNK, _chunk, 0)

    pltpu.sync_copy(den2.at[pl.ds(0, CH)], denS.at[iota_a], add=True)
    pltpu.sync_copy(den2.at[pl.ds(CH, DR - CH)], denS.at[iota_b], add=True)
    plsc.subcore_barrier()

    @pl.when(sid < 10)
    def _():
        pltpu.sync_copy(denS.at[pl.ds(sid * 16, 16)],
                        den_out.at[pl.ds(g * DR + sid * 16, 16)])


CHB = 96
NCHUNKB = EPT // CHB


@functools.partial(
    pl.kernel, mesh=_mesh, compiler_params=_params,
    out_type=jax.ShapeDtypeStruct((2 * NPAD, 128), jnp.float32),
    scratch_types=[
        pltpu.VMEM((CHB,), jnp.int32),
        pltpu.VMEM((CHB,), jnp.int32),
        pltpu.VMEM((CHB,), jnp.int32),
        pltpu.VMEM((CHB,), jnp.int32),
        pltpu.VMEM((CHB,), jnp.int32),
        pltpu.VMEM((CHB,), jnp.int32),
        pltpu.VMEM((2 * CHB,), jnp.float32),
        pltpu.VMEM((CHB, 128), jnp.float32),
        pltpu.VMEM((CHB, 128), jnp.float32),
        pltpu.VMEM((CHB, 128), jnp.float32),
        pltpu.VMEM_SHARED((NPAD, 128), jnp.float32),
        pltpu.SemaphoreType.DMA,
        pltpu.SemaphoreType.DMA,
        pltpu.SemaphoreType.DMA,
        pltpu.SemaphoreType.DMA,
        pltpu.SemaphoreType.DMA,
        pltpu.SemaphoreType.DMA,
    ],
)
def _sc_agg(hx2, srcoff, dst, wall, agg_out,
            so0, so1, so2, dv0, dv1, dv2, wv, g0, g1, g2, acc,
            gs0, gs1, gs2, ss0, ss1, ss2):
    g = lax.axis_index("c")
    sid = lax.axis_index("s")
    so = (so0, so1, so2)
    dv = (dv0, dv1, dv2)
    gb = (g0, g1, g2)
    gs = (gs0, gs1, gs2)
    ss = (ss0, ss1, ss2)

    def _z(i, _):
        g0[i >> 3, pl.ds((i & 7) * 16, 16)] = jnp.zeros((16,), jnp.float32)
        return 0
    lax.fori_loop(0, CHB * 8, _z, 0)
    for j in range(ROWS_PT // CHB):
        pltpu.sync_copy(g0, acc.at[pl.ds(sid * ROWS_PT + j * CHB, CHB)])
    rem = ROWS_PT - (ROWS_PT // CHB) * CHB
    pltpu.sync_copy(g0.at[pl.ds(0, rem)],
                    acc.at[pl.ds(sid * ROWS_PT + ROWS_PT - rem, rem)])
    plsc.subcore_barrier()

    ebase = g * EP + sid * EPT

    def _load_idx(c, j):
        off = ebase + c * CHB
        pltpu.sync_copy(srcoff.at[pl.ds(off, CHB)], so[j])
        pltpu.sync_copy(dst.at[pl.ds(off, CHB)], dv[j])

    def _wait_scatter(j):
        pltpu.make_async_copy(gb[j], acc.at[dv[j]], ss[j]).wait()

    _load_idx(0, 0)
    pltpu.async_copy(hx2.at[so[0]], gb[0], gs[0])

    def _body(i, _):
        for b in range(3):
            c = i * 3 + b
            nxt = (b + 1) % 3

            @pl.when(c >= 2)
            def _():
                _wait_scatter(nxt)

            @pl.when(c < NCHUNKB - 1)
            def _():
                _load_idx(c + 1, nxt)
                pltpu.async_copy(hx2.at[so[nxt]], gb[nxt], gs[nxt])

            offw = sid * EPT + c * CHB
            pltpu.sync_copy(wall.at[pl.ds(2 * g * EP + offw, CHB)],
                            wv.at[pl.ds(0, CHB)])
            pltpu.sync_copy(wall.at[pl.ds((2 * g + 1) * EP + offw, CHB)],
                            wv.at[pl.ds(CHB, CHB)])
            pltpu.make_async_copy(hx2.at[so[b]], gb[b], gs[b]).wait()

            gbb = gb[b]

            @plsc.parallel_loop(0, CHB, 1, unroll=8)
            def _mul(e):
                w0 = plsc.load_gather(wv, [jnp.full((16,), e, jnp.int32)])
                w1 = plsc.load_gather(wv, [jnp.full((16,), CHB + e, jnp.int32)])
                for d in range(4):
                    sl = pl.ds(d * 16, 16)
                    gbb[e, sl] = gbb[e, sl] * w0
                for d in range(4, 8):
                    sl = pl.ds(d * 16, 16)
                    gbb[e, sl] = gbb[e, sl] * w1

            pltpu.async_copy(gbb, acc.at[dv[b]], ss[b], add=True)
        return 0

    lax.fori_loop(0, NCHUNKB // 3, _body, 0)
    _wait_scatter((NCHUNKB - 2) % 3)
    _wait_scatter((NCHUNKB - 1) % 3)
    plsc.subcore_barrier()

    pltpu.sync_copy(acc.at[pl.ds(sid * ROWS_PT, ROWS_PT)],
                    agg_out.at[pl.ds(g * NPAD + sid * ROWS_PT, ROWS_PT)])


_TPT = TPAD // 16
_BPT = B // 16


@functools.partial(
    pl.kernel, mesh=_mesh, compiler_params=_params,
    out_type=[jax.ShapeDtypeStruct((2 * TPAD, 128), jnp.float32),
              jax.ShapeDtypeStruct((2 * B, 128), jnp.float32)],
    scratch_types=[
        pltpu.VMEM((CH,), jnp.int32),
        pltpu.VMEM((CH,), jnp.int32),
        pltpu.VMEM((CH,), jnp.int32),
        pltpu.VMEM((CH, 128), jnp.float32),
        pltpu.VMEM((CH, 128), jnp.float32),
        pltpu.VMEM((CH, 128), jnp.float32),
        pltpu.SemaphoreType.DMA,
        pltpu.SemaphoreType.DMA,
        pltpu.SemaphoreType.DMA,
    ],
)
def _sc_score(out2, rel2, ho, ro, to, bo, tr, bout,
              ho_v, ro_v, to_v, hb, rb, tb, s1, s2, s3):
    g = lax.axis_index("c")
    sid = lax.axis_index("s")
    tbase = g * TPAD + sid * _TPT

    def _chunk(c, _):
        off = tbase + c * CH
        pltpu.sync_copy(ho.at[pl.ds(off, CH)], ho_v)
        pltpu.sync_copy(ro.at[pl.ds(off, CH)], ro_v)
        pltpu.sync_copy(to.at[pl.ds(off, CH)], to_v)
        c1 = pltpu.async_copy(out2.at[ho_v], hb, s1)
        c2 = pltpu.async_copy(rel2.at[ro_v], rb, s2)
        c3 = pltpu.async_copy(out2.at[to_v], tb, s3)
        c1.wait(); c2.wait(); c3.wait()

        @plsc.parallel_loop(0, CH, 1, unroll=8)
        def _fuse(e):
            for d in range(8):
                sl = pl.ds(d * 16, 16)
                hb[e, sl] = hb[e, sl] + rb[e, sl] - tb[e, sl]
        pltpu.sync_copy(hb, tr.at[pl.ds(off, CH)])
        return 0
    lax.fori_loop(0, _TPT // CH, _chunk, 0)

    bbase = g * B + sid * _BPT

    def _bchunk(c, _):
        off = bbase + c * CH
        pltpu.sync_copy(bo.at[pl.ds(off, CH)], ho_v)
        pltpu.async_copy(out2.at[ho_v], hb, s1).wait()
        pltpu.sync_copy(hb, bout.at[pl.ds(off, CH)])
        return 0
    lax.fori_loop(0, _BPT // CH, _bchunk, 0)


def _pad_idx(x, total, spread):
    x = x.astype(jnp.int32)
    npad = total - x.shape[0]
    if spread:
        fill = N + (jnp.arange(npad, dtype=jnp.int32) % (NPAD - N))
    else:
        fill = jnp.zeros((npad,), jnp.int32)
    return jnp.concatenate([x, fill])


def _pack_logits(s):
    u = lax.bitcast_convert_type(s.astype(jnp.bfloat16), jnp.uint16)
    u = u.astype(jnp.uint32)
    psrc = (u[:, 0] | (u[:, 1] << 16)).astype(jnp.int32)
    pdst = (u[:, 2] | (u[:, 3] << 16)).astype(jnp.int32)
    psrc = psrc.reshape(2, NPAD)
    pdst = pdst.reshape(2, NPAD)
    return jnp.concatenate([psrc, pdst], axis=1)


def kernel(ent_sr, ent_tg, rel_sr, rel_tg, W, a, edge_sr, edge_tg,
           sr_data, tg_data, h_list_sr, h_list_tg, t_list_sr, t_list_tg,
           r_list_sr, r_list_tg):
    f32 = jnp.float32
    x0 = jnp.stack([jnp.pad(ent_sr.astype(f32), ((0, NPAD - N), (0, 0))),
                    jnp.pad(ent_tg.astype(f32), ((0, NPAD - N), (0, 0)))])
    x0 = x0.reshape(2 * NPAD, 128)

    Wcat = jnp.concatenate([W[:, 0], W[:, 1]], axis=-1).astype(f32)
    A = jnp.zeros((L, 128, 8), f32)
    A = A.at[:, :64, 0].set(a[:, 0, :64]).at[:, 64:, 1].set(a[:, 1, :64])
    A = A.at[:, :64, 2].set(a[:, 0, 64:]).at[:, 64:, 3].set(a[:, 1, 64:])

    goff = (jnp.arange(2, dtype=jnp.int32) * NPAD)[:, None]
    src2 = jnp.stack([_pad_idx(edge_sr[0], EP, True),
                      _pad_idx(edge_tg[0], EP, True)])
    dst2 = jnp.stack([_pad_idx(edge_sr[1], EP, True),
                      _pad_idx(edge_tg[1], EP, True)])
    srcoff = (src2 + goff).reshape(-1)
    src_f = src2.reshape(-1)
    dst_f = dst2.reshape(-1)

    def _den_t(d):
        return jnp.transpose(d.reshape(2, 2, NPAD), (0, 2, 1)).reshape(2 * NPAD, 2)

    hx, s = _tc_project(x0, Wcat[0], A[0])
    w01, den = _sc_att(_pack_logits(s), src_f, dst_f)
    agg = _sc_agg(hx, srcoff, dst_f, w01)

    hx, s = _tc_combine_project(agg, _den_t(den), Wcat[1], A[1])
    w01, den = _sc_att(_pack_logits(s), src_f, dst_f)
    agg = _sc_agg(hx, srcoff, dst_f, w01)

    out2 = _tc_final(agg, _den_t(den))

    rel2 = jnp.concatenate([rel_sr, rel_tg], axis=0).astype(f32)
    ho = jnp.concatenate([_pad_idx(h_list_sr, TPAD, False),
                          _pad_idx(h_list_tg, TPAD, False) + NPAD])
    to = jnp.concatenate([_pad_idx(t_list_sr, TPAD, False),
                          _pad_idx(t_list_tg, TPAD, False) + NPAD])
    ro = jnp.concatenate([_pad_idx(r_list_sr, TPAD, False),
                          _pad_idx(r_list_tg, TPAD, False) + R])
    bo = jnp.concatenate([sr_data.astype(jnp.int32),
                          tg_data.astype(jnp.int32) + NPAD])

    tr, bout = _sc_score(out2, rel2, ho, ro, to, bo)
    transe = jnp.concatenate([tr[:T], tr[TPAD:TPAD + T]], axis=0)
    return (bout[:B], bout[B:], transe)


# speedup vs baseline: 15.8858x; 1.0679x over previous
import functools

import jax
import jax.numpy as jnp
from jax import lax
from jax.experimental import pallas as pl
from jax.experimental.pallas import tpu as pltpu
from jax.experimental.pallas import tpu_sc as plsc

N = 10000
NPAD = 10240
DIM = 128
DH = 64
HEADS = 2
L = 2
R = 1000
T = 100000
TPAD = 102400
B = 4096
ALPHA = 0.2

E2 = 330000
CH = 128
NCHUNK = 162
EPT = NCHUNK * CH
EP = 16 * EPT
ROWS_PT = NPAD // 16
DR = 2 * NPAD // 128
BN = 1280

_mesh = plsc.VectorSubcoreMesh(core_axis_name="c", subcore_axis_name="s",
                               num_cores=2)
_params = pltpu.CompilerParams(needs_layout_passes=False)


def _tc_project(x, w, aa):
    def body(x_ref, w_ref, a_ref, hx_ref, s_ref):
        hx = jnp.dot(x_ref[...], w_ref[...], preferred_element_type=jnp.float32)
        hx_ref[...] = hx
        s_ref[...] = jnp.dot(hx, a_ref[...], preferred_element_type=jnp.float32)
    return pl.pallas_call(
        body,
        grid=(2 * NPAD // BN,),
        in_specs=[pl.BlockSpec((BN, 128), lambda i: (i, 0)),
                  pl.BlockSpec((128, 128), lambda i: (0, 0)),
                  pl.BlockSpec((128, 8), lambda i: (0, 0))],
        out_specs=[pl.BlockSpec((BN, 128), lambda i: (i, 0)),
                   pl.BlockSpec((BN, 8), lambda i: (i, 0))],
        out_shape=[jax.ShapeDtypeStruct((2 * NPAD, 128), jnp.float32),
                   jax.ShapeDtypeStruct((2 * NPAD, 8), jnp.float32)],
    )(x, w, aa)


def _combine(ag, dn):
    x0 = ag[:, 0:64] / dn[:, 0:1]
    x1 = ag[:, 64:128] / dn[:, 1:2]
    x = jnp.concatenate([x0, x1], axis=1)
    return jnp.where(x > 0, x, jnp.exp(x) - 1.0)


def _tc_combine_project(agg, den, w, aa):
    def body(g_ref, d_ref, w_ref, a_ref, hx_ref, s_ref):
        x = _combine(g_ref[...], d_ref[...])
        hx = jnp.dot(x, w_ref[...], preferred_element_type=jnp.float32)
        hx_ref[...] = hx
        s_ref[...] = jnp.dot(hx, a_ref[...], preferred_element_type=jnp.float32)
    return pl.pallas_call(
        body,
        grid=(2 * NPAD // BN,),
        in_specs=[pl.BlockSpec((BN, 128), lambda i: (i, 0)),
                  pl.BlockSpec((BN, 2), lambda i: (i, 0)),
                  pl.BlockSpec((128, 128), lambda i: (0, 0)),
                  pl.BlockSpec((128, 8), lambda i: (0, 0))],
        out_specs=[pl.BlockSpec((BN, 128), lambda i: (i, 0)),
                   pl.BlockSpec((BN, 8), lambda i: (i, 0))],
        out_shape=[jax.ShapeDtypeStruct((2 * NPAD, 128), jnp.float32),
                   jax.ShapeDtypeStruct((2 * NPAD, 8), jnp.float32)],
    )(agg, den, w, aa)


def _tc_final(agg, den):
    def body(g_ref, d_ref, o_ref):
        o_ref[...] = _combine(g_ref[...], d_ref[...])
    return pl.pallas_call(
        body,
        grid=(2 * NPAD // BN,),
        in_specs=[pl.BlockSpec((BN, 128), lambda i: (i, 0)),
                  pl.BlockSpec((BN, 2), lambda i: (i, 0))],
        out_specs=pl.BlockSpec((BN, 128), lambda i: (i, 0)),
        out_shape=jax.ShapeDtypeStruct((2 * NPAD, 128), jnp.float32),
    )(agg, den)


@functools.partial(
    pl.kernel, mesh=_mesh, compiler_params=_params,
    out_type=[jax.ShapeDtypeStruct((4 * EP,), jnp.float32),
              jax.ShapeDtypeStruct((2 * DR, 128), jnp.float32)],
    scratch_types=[
        pltpu.VMEM((2 * NPAD,), jnp.int32),
        pltpu.VMEM((CH,), jnp.int32),
        pltpu.VMEM((CH,), jnp.int32),
        pltpu.VMEM((2 * CH,), jnp.float32),
        pltpu.VMEM((DR, 128), jnp.float32),
        pltpu.VMEM((CH,), jnp.int32),
        pltpu.VMEM((32,), jnp.int32),
        pltpu.VMEM((16,), jnp.int32),
        pltpu.VMEM((16,), jnp.float32),
        pltpu.VMEM_SHARED((DR, 128), jnp.float32),
    ],
)
def _sc_att(stp, src, dst, wout, den_out,
            stv, src_v, dst_v, wtmp, den2, iota_a, iota_b, kbuf, mbuf, denS):
    g = lax.axis_index("c")
    sid = lax.axis_index("s")
    lane = lax.iota(jnp.int32, 16)

    pltpu.sync_copy(stp.at[g], stv)

    def _zd(i, _):
        den2[i >> 3, pl.ds((i & 7) * 16, 16)] = jnp.zeros((16,), jnp.float32)
        return 0
    lax.fori_loop(0, DR * 8, _zd, 0)

    def _zi(t, _):
        iota_a[pl.ds(t * 16, 16)] = lane + t * 16
        return 0
    lax.fori_loop(0, 8, _zi, 0)
    iota_b[pl.ds(0, 16)] = lane + CH
    iota_b[pl.ds(16, 16)] = lane + CH + 16

    @pl.when(sid == 0)
    def _():
        pltpu.sync_copy(den2.at[pl.ds(0, CH)], denS.at[pl.ds(0, CH)])
        pltpu.sync_copy(den2.at[pl.ds(0, DR - CH)], denS.at[pl.ds(CH, DR - CH)])

    plsc.subcore_barrier()

    mhi = jnp.full((16,), -65536, jnp.int32)

    def _chunk(c, _):
        off = sid * EPT + c * CH
        pltpu.sync_copy(src.at[pl.ds(g * EP + off, CH)], src_v)
        pltpu.sync_copy(dst.at[pl.ds(g * EP + off, CH)], dst_v)

        for t in range(CH // 16):
            sv = src_v[pl.ds(t * 16, 16)]
            dv = dst_v[pl.ds(t * 16, 16)]
            p_s = plsc.load_gather(stv, [sv])
            p_d = plsc.load_gather(stv, [dv + NPAD])
            e0 = (plsc.bitcast(lax.shift_left(p_s, 16), jnp.float32)
                  + plsc.bitcast(lax.shift_left(p_d, 16), jnp.float32))
            e1 = (plsc.bitcast(lax.bitwise_and(p_s, mhi), jnp.float32)
                  + plsc.bitcast(lax.bitwise_and(p_d, mhi), jnp.float32))
            e0 = jnp.where(e0 >= 0, e0, ALPHA * e0)
            e1 = jnp.where(e1 >= 0, e1, ALPHA * e1)
            w0 = jnp.exp(e0)
            w1 = jnp.exp(e1)
            wtmp[pl.ds(t * 16, 16)] = w0
            wtmp[pl.ds(CH + t * 16, 16)] = w1

            k, w0s = plsc.sort_key_val(dv, w0)
            _, w1s = plsc.sort_key_val(dv, w1)
            c0 = plsc.cumsum(w0s)
            c1 = plsc.cumsum(w1s)
            kbuf[pl.ds(0, 16)] = jnp.full((16,), -1, jnp.int32)
            plsc.store_scatter(kbuf, [lane - 1], k, mask=lane >= 1)
            isend = k != kbuf[pl.ds(0, 16)]
            row = lax.shift_right_logical(k, 7)
            col = lax.bitwise_and(k, 127)
            for cs, roff in ((c0, 0), (c1, NPAD >> 7)):
                m = jnp.where(isend, cs, 0.0)
                mbuf[pl.ds(0, 16)] = jnp.zeros((16,), jnp.float32)
                plsc.store_scatter(mbuf, [lane + 1], m, mask=lane <= 14)
                pe = plsc.cummax(mbuf[pl.ds(0, 16)])
                plsc.addupdate_scatter(den2, [row + roff, col], cs - pe,
                                       mask=isend)

        pltpu.sync_copy(wtmp.at[pl.ds(0, CH)],
                        wout.at[pl.ds(2 * g * EP + off, CH)])
        pltpu.sync_copy(wtmp.at[pl.ds(CH, CH)],
                        wout.at[pl.ds((2 * g + 1) * EP + off, CH)])
        return 0

    lax.fori_loop(0, NCHUNK, _chunk, 0)

    pltpu.sync_copy(den2.at[pl.ds(0, CH)], denS.at[iota_a], add=True)
    pltpu.sync_copy(den2.at[pl.ds(CH, DR - CH)], denS.at[iota_b], add=True)
    plsc.subcore_barrier()

    @pl.when(sid < 10)
    def _():
        pltpu.sync_copy(denS.at[pl.ds(sid * 16, 16)],
                        den_out.at[pl.ds(g * DR + sid * 16, 16)])


CHB = 96
NCHUNKB = EPT // CHB


@functools.partial(
    pl.kernel, mesh=_mesh, compiler_params=_params,
    out_type=jax.ShapeDtypeStruct((2 * NPAD, 128), jnp.float32),
    scratch_types=[
        pltpu.VMEM((CHB,), jnp.int32),
        pltpu.VMEM((CHB,), jnp.int32),
        pltpu.VMEM((CHB,), jnp.int32),
        pltpu.VMEM((CHB,), jnp.int32),
        pltpu.VMEM((CHB,), jnp.int32),
        pltpu.VMEM((CHB,), jnp.int32),
        pltpu.VMEM((2 * CHB,), jnp.float32),
        pltpu.VMEM((CHB, 128), jnp.float32),
        pltpu.VMEM((CHB, 128), jnp.float32),
        pltpu.VMEM((CHB, 128), jnp.float32),
        pltpu.VMEM_SHARED((NPAD, 128), jnp.float32),
        pltpu.SemaphoreType.DMA,
        pltpu.SemaphoreType.DMA,
        pltpu.SemaphoreType.DMA,
        pltpu.SemaphoreType.DMA,
        pltpu.SemaphoreType.DMA,
        pltpu.SemaphoreType.DMA,
    ],
)
def _sc_agg(hx2, srcoff, dst, wall, agg_out,
            so0, so1, so2, dv0, dv1, dv2, wv, g0, g1, g2, acc,
            gs0, gs1, gs2, ss0, ss1, ss2):
    g = lax.axis_index("c")
    sid = lax.axis_index("s")
    so = (so0, so1, so2)
    dv = (dv0, dv1, dv2)
    gb = (g0, g1, g2)
    gs = (gs0, gs1, gs2)
    ss = (ss0, ss1, ss2)

    def _z(i, _):
        g0[i >> 3, pl.ds((i & 7) * 16, 16)] = jnp.zeros((16,), jnp.float32)
        return 0
    lax.fori_loop(0, CHB * 8, _z, 0)
    for j in range(ROWS_PT // CHB):
        pltpu.sync_copy(g0, acc.at[pl.ds(sid * ROWS_PT + j * CHB, CHB)])
    rem = ROWS_PT - (ROWS_PT // CHB) * CHB
    pltpu.sync_copy(g0.at[pl.ds(0, rem)],
                    acc.at[pl.ds(sid * ROWS_PT + ROWS_PT - rem, rem)])
    plsc.subcore_barrier()

    ebase = g * EP + sid * EPT

    def _load_idx(c, j):
        off = ebase + c * CHB
        pltpu.sync_copy(srcoff.at[pl.ds(off, CHB)], so[j])
        pltpu.sync_copy(dst.at[pl.ds(off, CHB)], dv[j])

    def _wait_scatter(j):
        pltpu.make_async_copy(gb[j], acc.at[dv[j]], ss[j]).wait()

    _load_idx(0, 0)
    pltpu.async_copy(hx2.at[so[0]], gb[0], gs[0])

    def _body(i, _):
        for b in range(3):
            c = i * 3 + b
            nxt = (b + 1) % 3

            @pl.when(c >= 2)
            def _():
                _wait_scatter(nxt)

            @pl.when(c < NCHUNKB - 1)
            def _():
                _load_idx(c + 1, nxt)
                pltpu.async_copy(hx2.at[so[nxt]], gb[nxt], gs[nxt])

            offw = sid * EPT + c * CHB
            pltpu.sync_copy(wall.at[pl.ds(2 * g * EP + offw, CHB)],
                            wv.at[pl.ds(0, CHB)])
            pltpu.sync_copy(wall.at[pl.ds((2 * g + 1) * EP + offw, CHB)],
                            wv.at[pl.ds(CHB, CHB)])
            pltpu.make_async_copy(hx2.at[so[b]], gb[b], gs[b]).wait()

            gbb = gb[b]

            @plsc.parallel_loop(0, CHB, 1, unroll=8)
            def _mul(e):
                w0 = plsc.load_gather(wv, [jnp.full((16,), e, jnp.int32)])
                w1 = plsc.load_gather(wv, [jnp.full((16,), CHB + e, jnp.int32)])
                for d in range(4):
                    sl = pl.ds(d * 16, 16)
                    gbb[e, sl] = gbb[e, sl] * w0
                for d in range(4, 8):
                    sl = pl.ds(d * 16, 16)
                    gbb[e, sl] = gbb[e, sl] * w1

            pltpu.async_copy(gbb, acc.at[dv[b]], ss[b], add=True)
        return 0

    lax.fori_loop(0, NCHUNKB // 3, _body, 0)
    _wait_scatter((NCHUNKB - 2) % 3)
    _wait_scatter((NCHUNKB - 1) % 3)
    plsc.subcore_barrier()

    pltpu.sync_copy(acc.at[pl.ds(sid * ROWS_PT, ROWS_PT)],
                    agg_out.at[pl.ds(g * NPAD + sid * ROWS_PT, ROWS_PT)])


_TPT = TPAD // 16
_BPT = B // 16


@functools.partial(
    pl.kernel, mesh=_mesh, compiler_params=_params,
    out_type=[jax.ShapeDtypeStruct((2 * TPAD, 128), jnp.float32),
              jax.ShapeDtypeStruct((2 * B, 128), jnp.float32)],
    scratch_types=[
        pltpu.VMEM((CH,), jnp.int32),
        pltpu.VMEM((CH,), jnp.int32),
        pltpu.VMEM((CH,), jnp.int32),
        pltpu.VMEM((CH,), jnp.int32),
        pltpu.VMEM((CH,), jnp.int32),
        pltpu.VMEM((CH,), jnp.int32),
        pltpu.VMEM((CH, 128), jnp.float32),
        pltpu.VMEM((CH, 128), jnp.float32),
        pltpu.VMEM((CH, 128), jnp.float32),
        pltpu.VMEM((CH, 128), jnp.float32),
        pltpu.VMEM((CH, 128), jnp.float32),
        pltpu.VMEM((CH, 128), jnp.float32),
        pltpu.SemaphoreType.DMA,
        pltpu.SemaphoreType.DMA,
        pltpu.SemaphoreType.DMA,
        pltpu.SemaphoreType.DMA,
        pltpu.SemaphoreType.DMA,
        pltpu.SemaphoreType.DMA,
        pltpu.SemaphoreType.DMA,
        pltpu.SemaphoreType.DMA,
    ],
)
def _sc_score(out2, rel2, ho, ro, to, bo, tr, bout,
              h0, r0, t0, h1, r1, t1, hb0, rb0, tb0, hb1, rb1, tb1,
              g10, g20, g30, g11, g21, g31, os0, os1):
    g = lax.axis_index("c")
    sid = lax.axis_index("s")
    tbase = g * TPAD + sid * _TPT
    NC = _TPT // CH
    hv = (h0, h1)
    rv = (r0, r1)
    tv = (t0, t1)
    hb = (hb0, hb1)
    rb = (rb0, rb1)
    tb = (tb0, tb1)
    gsm = ((g10, g20, g30), (g11, g21, g31))
    osm = (os0, os1)

    def _load_idx(c, j):
        off = tbase + c * CH
        pltpu.sync_copy(ho.at[pl.ds(off, CH)], hv[j])
        pltpu.sync_copy(ro.at[pl.ds(off, CH)], rv[j])
        pltpu.sync_copy(to.at[pl.ds(off, CH)], tv[j])

    def _start_gathers(j):
        pltpu.async_copy(out2.at[hv[j]], hb[j], gsm[j][0])
        pltpu.async_copy(rel2.at[rv[j]], rb[j], gsm[j][1])
        pltpu.async_copy(out2.at[tv[j]], tb[j], gsm[j][2])

    def _wait_gathers(j):
        pltpu.make_async_copy(out2.at[hv[j]], hb[j], gsm[j][0]).wait()
        pltpu.make_async_copy(rel2.at[rv[j]], rb[j], gsm[j][1]).wait()
        pltpu.make_async_copy(out2.at[tv[j]], tb[j], gsm[j][2]).wait()

    def _wait_store(c, j):
        pltpu.make_async_copy(hb[j], tr.at[pl.ds(tbase + c * CH, CH)],
                              osm[j]).wait()

    _load_idx(0, 0)
    _start_gathers(0)

    def _body(i, _):
        for b in range(2):
            c = i * 2 + b
            nxt = 1 - b

            @pl.when(c >= 1)
            def _():
                _wait_store(c - 1, nxt)

            @pl.when(c < NC - 1)
            def _():
                _load_idx(c + 1, nxt)
                _start_gathers(nxt)

            _wait_gathers(b)
            hbb, rbb, tbb = hb[b], rb[b], tb[b]

            @plsc.parallel_loop(0, CH, 1, unroll=8)
            def _fuse(e):
                for d in range(8):
                    sl = pl.ds(d * 16, 16)
                    hbb[e, sl] = hbb[e, sl] + rbb[e, sl] - tbb[e, sl]

            pltpu.async_copy(hbb, tr.at[pl.ds(tbase + c * CH, CH)], osm[b])
        return 0

    lax.fori_loop(0, NC // 2, _body, 0)
    _wait_store(NC - 1, (NC - 1) % 2)

    bbase = g * B + sid * _BPT

    def _bchunk(c, _):
        off = bbase + c * CH
        pltpu.sync_copy(bo.at[pl.ds(off, CH)], h0)
        pltpu.async_copy(out2.at[h0], hb0, g10).wait()
        pltpu.sync_copy(hb0, bout.at[pl.ds(off, CH)])
        return 0
    lax.fori_loop(0, _BPT // CH, _bchunk, 0)


def _pad_idx(x, total, spread):
    x = x.astype(jnp.int32)
    npad = total - x.shape[0]
    if spread:
        fill = N + (jnp.arange(npad, dtype=jnp.int32) % (NPAD - N))
    else:
        fill = jnp.zeros((npad,), jnp.int32)
    return jnp.concatenate([x, fill])


def _pack_logits(s):
    u = lax.bitcast_convert_type(s.astype(jnp.bfloat16), jnp.uint16)
    u = u.astype(jnp.uint32)
    psrc = (u[:, 0] | (u[:, 1] << 16)).astype(jnp.int32)
    pdst = (u[:, 2] | (u[:, 3] << 16)).astype(jnp.int32)
    psrc = psrc.reshape(2, NPAD)
    pdst = pdst.reshape(2, NPAD)
    return jnp.concatenate([psrc, pdst], axis=1)


def kernel(ent_sr, ent_tg, rel_sr, rel_tg, W, a, edge_sr, edge_tg,
           sr_data, tg_data, h_list_sr, h_list_tg, t_list_sr, t_list_tg,
           r_list_sr, r_list_tg):
    f32 = jnp.float32
    x0 = jnp.stack([jnp.pad(ent_sr.astype(f32), ((0, NPAD - N), (0, 0))),
                    jnp.pad(ent_tg.astype(f32), ((0, NPAD - N), (0, 0)))])
    x0 = x0.reshape(2 * NPAD, 128)

    Wcat = jnp.concatenate([W[:, 0], W[:, 1]], axis=-1).astype(f32)
    A = jnp.zeros((L, 128, 8), f32)
    A = A.at[:, :64, 0].set(a[:, 0, :64]).at[:, 64:, 1].set(a[:, 1, :64])
    A = A.at[:, :64, 2].set(a[:, 0, 64:]).at[:, 64:, 3].set(a[:, 1, 64:])

    goff = (jnp.arange(2, dtype=jnp.int32) * NPAD)[:, None]
    src2 = jnp.stack([_pad_idx(edge_sr[0], EP, True),
                      _pad_idx(edge_tg[0], EP, True)])
    dst2 = jnp.stack([_pad_idx(edge_sr[1], EP, True),
                      _pad_idx(edge_tg[1], EP, True)])
    srcoff = (src2 + goff).reshape(-1)
    src_f = src2.reshape(-1)
    dst_f = dst2.reshape(-1)

    def _den_t(d):
        return jnp.transpose(d.reshape(2, 2, NPAD), (0, 2, 1)).reshape(2 * NPAD, 2)

    hx, s = _tc_project(x0, Wcat[0], A[0])
    w01, den = _sc_att(_pack_logits(s), src_f, dst_f)
    agg = _sc_agg(hx, srcoff, dst_f, w01)

    hx, s = _tc_combine_project(agg, _den_t(den), Wcat[1], A[1])
    w01, den = _sc_att(_pack_logits(s), src_f, dst_f)
    agg = _sc_agg(hx, srcoff, dst_f, w01)

    out2 = _tc_final(agg, _den_t(den))

    rel2 = jnp.concatenate([rel_sr, rel_tg], axis=0).astype(f32)
    ho = jnp.concatenate([_pad_idx(h_list_sr, TPAD, False),
                          _pad_idx(h_list_tg, TPAD, False) + NPAD])
    to = jnp.concatenate([_pad_idx(t_list_sr, TPAD, False),
                          _pad_idx(t_list_tg, TPAD, False) + NPAD])
    ro = jnp.concatenate([_pad_idx(r_list_sr, TPAD, False),
                          _pad_idx(r_list_tg, TPAD, False) + R])
    bo = jnp.concatenate([sr_data.astype(jnp.int32),
                          tg_data.astype(jnp.int32) + NPAD])

    tr, bout = _sc_score(out2, rel2, ho, ro, to, bo)
    transe = jnp.concatenate([tr[:T], tr[TPAD:TPAD + T]], axis=0)
    return (bout[:B], bout[B:], transe)
